# baseline (speedup 1.0000x reference)
import functools

import jax
import jax.numpy as jnp
from jax import lax
from jax.experimental import pallas as pl
from jax.experimental.pallas import tpu as pltpu
from jax.experimental.pallas import tpu_sc as plsc

E = 8
TOPK = 2
D = 768
F = 2048
T = 4096
NPAIR = T * TOPK

TM = 128
NTILES = NPAIR // TM + E
PP = NTILES * TM

BC = 128
NB = T // BC

NC = 2
NS = 16
NW = NC * NS
TOK_W = T // NW
CT = 64
NV = D // 16


def _router_body(x_ref, wg_ref, bg_ref, p0_ref, p1_ref, w0_ref, w1_ref, g_ref,
                 m_scr, cs_scr):
    x = x_ref[...]
    logits = jnp.dot(x, wg_ref[...], preferred_element_type=jnp.float32)
    logits = logits + bg_ref[...]
    mx = jnp.max(logits, axis=1, keepdims=True)
    ex = jnp.exp(logits - mx)
    probs = ex / jnp.sum(ex, axis=1, keepdims=True)

    iota = lax.broadcasted_iota(jnp.int32, (T, E), 1)
    w0 = jnp.max(probs, axis=1, keepdims=True)
    i0 = jnp.min(jnp.where(probs == w0, iota, E), axis=1, keepdims=True)
    probs2 = jnp.where(iota == i0, -jnp.inf, probs)
    w1 = jnp.max(probs2, axis=1, keepdims=True)
    i1 = jnp.min(jnp.where(probs2 == w1, iota, E), axis=1, keepdims=True)
    s = w0 + w1
    w0 = w0 / s
    w1 = w1 / s

    m_scr[...] = ((iota == i0) | (iota == i1)).astype(jnp.float32)

    r = lax.broadcasted_iota(jnp.int32, (BC, BC), 0)
    c = lax.broadcasted_iota(jnp.int32, (BC, BC), 1)
    ltri = (c < r).astype(jnp.float32)

    def blk(b, off):
        rows = m_scr[pl.ds(b * BC, BC), :]
        cs_scr[pl.ds(b * BC, BC), :] = (
            jnp.dot(ltri, rows, preferred_element_type=jnp.float32) + off
        )
        return off + jnp.sum(rows, axis=0, keepdims=True)

    cnt = lax.fori_loop(0, NB, blk, jnp.zeros((1, E), jnp.float32))

    pc = jnp.ceil(cnt / TM) * TM
    re8 = lax.broadcasted_iota(jnp.int32, (E, E), 0)
    ce8 = lax.broadcasted_iota(jnp.int32, (E, E), 1)
    ut8 = (re8 < ce8).astype(jnp.float32)
    po = jnp.dot(pc, ut8, preferred_element_type=jnp.float32)

    base = po + cs_scr[...]
    pos0 = jnp.sum(jnp.where(iota == i0, base, 0.0), axis=1, keepdims=True)
    pos1 = jnp.sum(jnp.where(iota == i1, base, 0.0), axis=1, keepdims=True)

    p0_ref[...] = pos0.astype(jnp.int32)
    p1_ref[...] = pos1.astype(jnp.int32)
    w0_ref[...] = w0
    w1_ref[...] = w1

    jt = lax.broadcasted_iota(jnp.int32, (NTILES, E), 0).astype(jnp.float32) * TM
    ends = po + pc
    gmap = jnp.sum((jt >= ends).astype(jnp.int32), axis=1, keepdims=True)
    g_ref[...] = jnp.minimum(gmap, E - 1)


def _router(x, Wg, bg):
    return pl.pallas_call(
        _router_body,
        out_shape=(
            jax.ShapeDtypeStruct((T, 1), jnp.int32),
            jax.ShapeDtypeStruct((T, 1), jnp.int32),
            jax.ShapeDtypeStruct((T, 1), jnp.float32),
            jax.ShapeDtypeStruct((T, 1), jnp.float32),
            jax.ShapeDtypeStruct((NTILES, 1), jnp.int32),
        ),
        scratch_shapes=[
            pltpu.VMEM((T, E), jnp.float32),
            pltpu.VMEM((T, E), jnp.float32),
        ],
    )(x, Wg, bg.reshape(1, E))


@functools.cache
def _sc_kernels():
    mesh = plsc.VectorSubcoreMesh(
        core_axis_name="c", subcore_axis_name="s", num_cores=NC, num_subcores=NS
    )

    @functools.partial(
        pl.kernel,
        out_type=(
            jax.ShapeDtypeStruct((PP, D), jnp.float32),
            jax.ShapeDtypeStruct((PP,), jnp.float32),
        ),
        mesh=mesh,
        scratch_types=[
            pltpu.VMEM((TOK_W, D), jnp.float32),
            pltpu.VMEM((TOK_W,), jnp.int32),
            pltpu.VMEM((TOK_W,), jnp.int32),
            pltpu.VMEM((TOK_W,), jnp.float32),
            pltpu.VMEM((TOK_W,), jnp.float32),
            pltpu.SemaphoreType.DMA,
            pltpu.SemaphoreType.DMA,
        ],
    )
    def dispatch_kernel(x_hbm, p0_hbm, p1_hbm, w0_hbm, w1_hbm,
                        xg_hbm, ws_hbm, xv, i0v, i1v, wv0, wv1, s0, s1):
        wid = lax.axis_index("s") * NC + lax.axis_index("c")
        base = wid * TOK_W
        lx = pltpu.async_copy(x_hbm.at[pl.ds(base, TOK_W)], xv, s0)
        l0 = pltpu.async_copy(p0_hbm.at[pl.ds(base, TOK_W)], i0v, s1)
        l1 = pltpu.async_copy(p1_hbm.at[pl.ds(base, TOK_W)], i1v, s1)
        lw0 = pltpu.async_copy(w0_hbm.at[pl.ds(base, TOK_W)], wv0, s1)
        lw1 = pltpu.async_copy(w1_hbm.at[pl.ds(base, TOK_W)], wv1, s1)
        lx.wait()
        l0.wait()
        l1.wait()
        lw0.wait()
        lw1.wait()
        c0 = pltpu.async_copy(xv, xg_hbm.at[i0v], s0)
        c1 = pltpu.async_copy(xv, xg_hbm.at[i1v], s1)
        cw0 = pltpu.async_copy(wv0, ws_hbm.at[i0v], s0)
        cw1 = pltpu.async_copy(wv1, ws_hbm.at[i1v], s1)
        c0.wait()
        c1.wait()
        cw0.wait()
        cw1.wait()

    @functools.partial(
        pl.kernel,
        out_type=jax.ShapeDtypeStruct((T, D), jnp.float32),
        mesh=mesh,
        scratch_types=[
            pltpu.VMEM((CT,), jnp.int32),
            pltpu.VMEM((CT,), jnp.int32),
            pltpu.VMEM((CT, D), jnp.float32),
            pltpu.VMEM((CT, D), jnp.float32),
            pltpu.SemaphoreType.DMA,
            pltpu.SemaphoreType.DMA,
        ],
    )
    def combine_kernel(y_hbm, p0_hbm, p1_hbm, out_hbm, i0v, i1v, b0, b1, s0, s1):
        wid = lax.axis_index("s") * NC + lax.axis_index("c")
        for cchunk in range(TOK_W // CT):
            off = wid * TOK_W + cchunk * CT
            pltpu.sync_copy(p0_hbm.at[pl.ds(off, CT)], i0v)
            pltpu.sync_copy(p1_hbm.at[pl.ds(off, CT)], i1v)
            c0 = pltpu.async_copy(y_hbm.at[i0v], b0, s0)
            c1 = pltpu.async_copy(y_hbm.at[i1v], b1, s1)
            c0.wait()
            c1.wait()

            def row_add(r, carry):
                for j in range(NV):
                    sl = pl.ds(j * 16, 16)
                    b0[r, sl] = b0[r, sl] + b1[r, sl]
                return carry

            lax.fori_loop(0, CT, row_add, 0)
            pltpu.sync_copy(b0, out_hbm.at[pl.ds(off, CT)])

    return dispatch_kernel, combine_kernel


def _mlp_body(g_ref, xg_ref, ws_ref, w1_ref, w3_ref, w2_ref, y_ref):
    xb = xg_ref[...]
    h1 = jnp.dot(xb, w1_ref[0], preferred_element_type=jnp.float32,
                 precision=lax.Precision.DEFAULT)
    h3 = jnp.dot(xb, w3_ref[0], preferred_element_type=jnp.float32,
                 precision=lax.Precision.DEFAULT)
    h = (h1 * jax.nn.sigmoid(h1)) * h3
    y = jnp.dot(h, w2_ref[0], preferred_element_type=jnp.float32,
                precision=lax.Precision.DEFAULT)
    y_ref[...] = y * ws_ref[0]


def _grouped_mlp(g, Xg, ws, W1, W3, W2):
    grid_spec = pltpu.PrefetchScalarGridSpec(
        num_scalar_prefetch=1,
        grid=(NTILES,),
        in_specs=[
            pl.BlockSpec((TM, D), lambda j, gm: (j, 0)),
            pl.BlockSpec((1, TM, 1), lambda j, gm: (j, 0, 0)),
            pl.BlockSpec((1, D, F), lambda j, gm: (gm[j], 0, 0)),
            pl.BlockSpec((1, D, F), lambda j, gm: (gm[j], 0, 0)),
            pl.BlockSpec((1, F, D), lambda j, gm: (gm[j], 0, 0)),
        ],
        out_specs=pl.BlockSpec((TM, D), lambda j, gm: (j, 0)),
    )
    return pl.pallas_call(
        _mlp_body,
        grid_spec=grid_spec,
        out_shape=jax.ShapeDtypeStruct((PP, D), jnp.float32),
    )(g, Xg, ws.reshape(NTILES, TM, 1), W1, W3, W2)


def kernel(x, Wg, bg, W1, W3, W2):
    dispatch_kernel, combine_kernel = _sc_kernels()
    p0, p1, w0, w1, g = _router(x, Wg, bg)
    p0 = p0.reshape(T)
    p1 = p1.reshape(T)
    Xg, ws = dispatch_kernel(x, p0, p1, w0.reshape(T), w1.reshape(T))
    Y = _grouped_mlp(g.reshape(NTILES), Xg, ws, W1, W3, W2)
    return combine_kernel(Y, p0, p1)


# speedup vs baseline: 1.1161x; 1.1161x over previous
import functools

import jax
import jax.numpy as jnp
from jax import lax
from jax.experimental import pallas as pl
from jax.experimental.pallas import tpu as pltpu
from jax.experimental.pallas import tpu_sc as plsc

E = 8
TOPK = 2
D = 768
F = 2048
T = 4096
NPAIR = T * TOPK

TM = 256
NTILES = NPAIR // TM + E
PP = NTILES * TM

BC = 128
NB = T // BC

NC = 2
NS = 16
NW = NC * NS
TOK_W = T // NW
CT = 64
NV = D // 16


def _router_body(x_ref, wg_ref, bg_ref, p0_ref, p1_ref, w0_ref, w1_ref, g_ref,
                 m_scr, cs_scr):
    x = x_ref[...]
    logits = jnp.dot(x, wg_ref[...], preferred_element_type=jnp.float32)
    logits = logits + bg_ref[...]
    mx = jnp.max(logits, axis=1, keepdims=True)
    ex = jnp.exp(logits - mx)
    probs = ex / jnp.sum(ex, axis=1, keepdims=True)

    iota = lax.broadcasted_iota(jnp.int32, (T, E), 1)
    w0 = jnp.max(probs, axis=1, keepdims=True)
    i0 = jnp.min(jnp.where(probs == w0, iota, E), axis=1, keepdims=True)
    probs2 = jnp.where(iota == i0, -jnp.inf, probs)
    w1 = jnp.max(probs2, axis=1, keepdims=True)
    i1 = jnp.min(jnp.where(probs2 == w1, iota, E), axis=1, keepdims=True)
    s = w0 + w1
    w0 = w0 / s
    w1 = w1 / s

    m_scr[...] = ((iota == i0) | (iota == i1)).astype(jnp.float32)

    r = lax.broadcasted_iota(jnp.int32, (BC, BC), 0)
    c = lax.broadcasted_iota(jnp.int32, (BC, BC), 1)
    ltri = (c < r).astype(jnp.float32)

    def blk(b, off):
        rows = m_scr[pl.ds(b * BC, BC), :]
        cs_scr[pl.ds(b * BC, BC), :] = (
            jnp.dot(ltri, rows, preferred_element_type=jnp.float32) + off
        )
        return off + jnp.sum(rows, axis=0, keepdims=True)

    cnt = lax.fori_loop(0, NB, blk, jnp.zeros((1, E), jnp.float32))

    pc = jnp.ceil(cnt / TM) * TM
    re8 = lax.broadcasted_iota(jnp.int32, (E, E), 0)
    ce8 = lax.broadcasted_iota(jnp.int32, (E, E), 1)
    ut8 = (re8 < ce8).astype(jnp.float32)
    po = jnp.dot(pc, ut8, preferred_element_type=jnp.float32)

    base = po + cs_scr[...]
    pos0 = jnp.sum(jnp.where(iota == i0, base, 0.0), axis=1, keepdims=True)
    pos1 = jnp.sum(jnp.where(iota == i1, base, 0.0), axis=1, keepdims=True)

    p0_ref[...] = pos0.astype(jnp.int32).reshape(T // 128, 128)
    p1_ref[...] = pos1.astype(jnp.int32).reshape(T // 128, 128)
    w0_ref[...] = w0.reshape(T // 128, 128)
    w1_ref[...] = w1.reshape(T // 128, 128)

    jt = lax.broadcasted_iota(jnp.int32, (NTILES, E), 0).astype(jnp.float32) * TM
    ends = po + pc
    gmap = jnp.sum((jt >= ends).astype(jnp.int32), axis=1, keepdims=True)
    g_ref[...] = jnp.minimum(gmap, E - 1)


def _router(x, Wg, bg):
    return pl.pallas_call(
        _router_body,
        out_shape=(
            jax.ShapeDtypeStruct((T // 128, 128), jnp.int32),
            jax.ShapeDtypeStruct((T // 128, 128), jnp.int32),
            jax.ShapeDtypeStruct((T // 128, 128), jnp.float32),
            jax.ShapeDtypeStruct((T // 128, 128), jnp.float32),
            jax.ShapeDtypeStruct((NTILES, 1), jnp.int32),
        ),
        scratch_shapes=[
            pltpu.VMEM((T, E), jnp.float32),
            pltpu.VMEM((T, E), jnp.float32),
        ],
    )(x, Wg, bg.reshape(1, E))


@functools.cache
def _sc_kernels():
    mesh = plsc.VectorSubcoreMesh(
        core_axis_name="c", subcore_axis_name="s", num_cores=NC, num_subcores=NS
    )

    @functools.partial(
        pl.kernel,
        out_type=(
            jax.ShapeDtypeStruct((PP, D), jnp.float32),
            jax.ShapeDtypeStruct((PP,), jnp.float32),
        ),
        mesh=mesh,
        scratch_types=[
            pltpu.VMEM((TOK_W, D), jnp.float32),
            pltpu.VMEM((TOK_W,), jnp.int32),
            pltpu.VMEM((TOK_W,), jnp.int32),
            pltpu.VMEM((TOK_W,), jnp.float32),
            pltpu.VMEM((TOK_W,), jnp.float32),
            pltpu.SemaphoreType.DMA,
            pltpu.SemaphoreType.DMA,
        ],
    )
    def dispatch_kernel(x_hbm, p0_hbm, p1_hbm, w0_hbm, w1_hbm,
                        xg_hbm, ws_hbm, xv, i0v, i1v, wv0, wv1, s0, s1):
        wid = lax.axis_index("s") * NC + lax.axis_index("c")
        base = wid * TOK_W
        lx = pltpu.async_copy(x_hbm.at[pl.ds(base, TOK_W)], xv, s0)
        l0 = pltpu.async_copy(p0_hbm.at[pl.ds(base, TOK_W)], i0v, s1)
        l1 = pltpu.async_copy(p1_hbm.at[pl.ds(base, TOK_W)], i1v, s1)
        lw0 = pltpu.async_copy(w0_hbm.at[pl.ds(base, TOK_W)], wv0, s1)
        lw1 = pltpu.async_copy(w1_hbm.at[pl.ds(base, TOK_W)], wv1, s1)
        lx.wait()
        l0.wait()
        l1.wait()
        lw0.wait()
        lw1.wait()
        c0 = pltpu.async_copy(xv, xg_hbm.at[i0v], s0)
        c1 = pltpu.async_copy(xv, xg_hbm.at[i1v], s1)
        cw0 = pltpu.async_copy(wv0, ws_hbm.at[i0v], s0)
        cw1 = pltpu.async_copy(wv1, ws_hbm.at[i1v], s1)
        c0.wait()
        c1.wait()
        cw0.wait()
        cw1.wait()

    @functools.partial(
        pl.kernel,
        out_type=jax.ShapeDtypeStruct((T, D), jnp.float32),
        mesh=mesh,
        scratch_types=[
            pltpu.VMEM((CT,), jnp.int32),
            pltpu.VMEM((CT,), jnp.int32),
            pltpu.VMEM((CT, D), jnp.float32),
            pltpu.VMEM((CT, D), jnp.float32),
            pltpu.SemaphoreType.DMA,
            pltpu.SemaphoreType.DMA,
        ],
    )
    def combine_kernel(y_hbm, p0_hbm, p1_hbm, out_hbm, i0v, i1v, b0, b1, s0, s1):
        wid = lax.axis_index("s") * NC + lax.axis_index("c")
        for cchunk in range(TOK_W // CT):
            off = wid * TOK_W + cchunk * CT
            pltpu.sync_copy(p0_hbm.at[pl.ds(off, CT)], i0v)
            pltpu.sync_copy(p1_hbm.at[pl.ds(off, CT)], i1v)
            c0 = pltpu.async_copy(y_hbm.at[i0v], b0, s0)
            c1 = pltpu.async_copy(y_hbm.at[i1v], b1, s1)
            c0.wait()
            c1.wait()

            def row_add(r, carry):
                for j in range(NV):
                    sl = pl.ds(j * 16, 16)
                    b0[r, sl] = b0[r, sl] + b1[r, sl]
                return carry

            lax.fori_loop(0, CT, row_add, 0)
            pltpu.sync_copy(b0, out_hbm.at[pl.ds(off, CT)])

    return dispatch_kernel, combine_kernel


def _mlp_body(g_ref, xg_ref, ws_ref, w1_ref, w3_ref, w2_ref, y_ref):
    xb = xg_ref[...]
    h1 = jnp.dot(xb, w1_ref[0], preferred_element_type=jnp.float32,
                 precision=lax.Precision.DEFAULT)
    h3 = jnp.dot(xb, w3_ref[0], preferred_element_type=jnp.float32,
                 precision=lax.Precision.DEFAULT)
    h = (h1 * jax.nn.sigmoid(h1)) * h3
    y = jnp.dot(h, w2_ref[0], preferred_element_type=jnp.float32,
                precision=lax.Precision.DEFAULT)
    y_ref[...] = y * ws_ref[0]


def _grouped_mlp(g, Xg, ws, W1, W3, W2):
    grid_spec = pltpu.PrefetchScalarGridSpec(
        num_scalar_prefetch=1,
        grid=(NTILES,),
        in_specs=[
            pl.BlockSpec((TM, D), lambda j, gm: (j, 0)),
            pl.BlockSpec((1, TM, 1), lambda j, gm: (j, 0, 0)),
            pl.BlockSpec((1, D, F), lambda j, gm: (gm[j], 0, 0)),
            pl.BlockSpec((1, D, F), lambda j, gm: (gm[j], 0, 0)),
            pl.BlockSpec((1, F, D), lambda j, gm: (gm[j], 0, 0)),
        ],
        out_specs=pl.BlockSpec((TM, D), lambda j, gm: (j, 0)),
    )
    return pl.pallas_call(
        _mlp_body,
        grid_spec=grid_spec,
        out_shape=jax.ShapeDtypeStruct((PP, D), jnp.float32),
    )(g, Xg, ws.reshape(NTILES, TM, 1), W1, W3, W2)


def kernel(x, Wg, bg, W1, W3, W2):
    dispatch_kernel, combine_kernel = _sc_kernels()
    p0, p1, w0, w1, g = _router(x, Wg, bg)
    p0 = p0.reshape(T)
    p1 = p1.reshape(T)
    Xg, ws = dispatch_kernel(x, p0, p1, w0.reshape(T), w1.reshape(T))
    Y = _grouped_mlp(g.reshape(NTILES), Xg, ws, W1, W3, W2)
    return combine_kernel(Y, p0, p1)


# speedup vs baseline: 1.1233x; 1.0065x over previous
import functools

import jax
import jax.numpy as jnp
from jax import lax
from jax.experimental import pallas as pl
from jax.experimental.pallas import tpu as pltpu
from jax.experimental.pallas import tpu_sc as plsc

E = 8
TOPK = 2
D = 768
F = 2048
T = 4096
NPAIR = T * TOPK

TM = 256
NTILES = NPAIR // TM + E
PP = NTILES * TM

BC = 128
NB = T // BC

NC = 2
NS = 16
NW = NC * NS
TOK_W = T // NW
CT = 32
NV = D // 16


def _router_body(x_ref, wg_ref, bg_ref, p0_ref, p1_ref, w0_ref, w1_ref, g_ref,
                 m_scr, cs_scr):
    x = x_ref[...]
    logits = jnp.dot(x, wg_ref[...], preferred_element_type=jnp.float32)
    logits = logits + bg_ref[...]
    mx = jnp.max(logits, axis=1, keepdims=True)
    ex = jnp.exp(logits - mx)
    probs = ex / jnp.sum(ex, axis=1, keepdims=True)

    iota = lax.broadcasted_iota(jnp.int32, (T, E), 1)
    w0 = jnp.max(probs, axis=1, keepdims=True)
    i0 = jnp.min(jnp.where(probs == w0, iota, E), axis=1, keepdims=True)
    probs2 = jnp.where(iota == i0, -jnp.inf, probs)
    w1 = jnp.max(probs2, axis=1, keepdims=True)
    i1 = jnp.min(jnp.where(probs2 == w1, iota, E), axis=1, keepdims=True)
    s = w0 + w1
    w0 = w0 / s
    w1 = w1 / s

    m_scr[...] = ((iota == i0) | (iota == i1)).astype(jnp.float32)

    r = lax.broadcasted_iota(jnp.int32, (BC, BC), 0)
    c = lax.broadcasted_iota(jnp.int32, (BC, BC), 1)
    ltri = (c < r).astype(jnp.float32)

    def blk(b, off):
        rows = m_scr[pl.ds(b * BC, BC), :]
        cs_scr[pl.ds(b * BC, BC), :] = (
            jnp.dot(ltri, rows, preferred_element_type=jnp.float32) + off
        )
        return off + jnp.sum(rows, axis=0, keepdims=True)

    cnt = lax.fori_loop(0, NB, blk, jnp.zeros((1, E), jnp.float32))

    pc = jnp.ceil(cnt / TM) * TM
    re8 = lax.broadcasted_iota(jnp.int32, (E, E), 0)
    ce8 = lax.broadcasted_iota(jnp.int32, (E, E), 1)
    ut8 = (re8 < ce8).astype(jnp.float32)
    po = jnp.dot(pc, ut8, preferred_element_type=jnp.float32)

    base = po + cs_scr[...]
    pos0 = jnp.sum(jnp.where(iota == i0, base, 0.0), axis=1, keepdims=True)
    pos1 = jnp.sum(jnp.where(iota == i1, base, 0.0), axis=1, keepdims=True)

    p0_ref[...] = pos0.astype(jnp.int32).reshape(T // 128, 128)
    p1_ref[...] = pos1.astype(jnp.int32).reshape(T // 128, 128)
    w0_ref[...] = w0.reshape(T // 128, 128)
    w1_ref[...] = w1.reshape(T // 128, 128)

    jt = lax.broadcasted_iota(jnp.int32, (NTILES, E), 0).astype(jnp.float32) * TM
    ends = po + pc
    gmap = jnp.sum((jt >= ends).astype(jnp.int32), axis=1, keepdims=True)
    g_ref[...] = jnp.minimum(gmap, E - 1)


def _router(x, Wg, bg):
    return pl.pallas_call(
        _router_body,
        out_shape=(
            jax.ShapeDtypeStruct((T // 128, 128), jnp.int32),
            jax.ShapeDtypeStruct((T // 128, 128), jnp.int32),
            jax.ShapeDtypeStruct((T // 128, 128), jnp.float32),
            jax.ShapeDtypeStruct((T // 128, 128), jnp.float32),
            jax.ShapeDtypeStruct((NTILES, 1), jnp.int32),
        ),
        scratch_shapes=[
            pltpu.VMEM((T, E), jnp.float32),
            pltpu.VMEM((T, E), jnp.float32),
        ],
    )(x, Wg, bg.reshape(1, E))


@functools.cache
def _sc_kernels():
    mesh = plsc.VectorSubcoreMesh(
        core_axis_name="c", subcore_axis_name="s", num_cores=NC, num_subcores=NS
    )

    @functools.partial(
        pl.kernel,
        out_type=(
            jax.ShapeDtypeStruct((PP, D), jnp.float32),
            jax.ShapeDtypeStruct((PP,), jnp.float32),
        ),
        mesh=mesh,
        scratch_types=[
            pltpu.VMEM((TOK_W, D), jnp.float32),
            pltpu.VMEM((TOK_W,), jnp.int32),
            pltpu.VMEM((TOK_W,), jnp.int32),
            pltpu.VMEM((TOK_W,), jnp.float32),
            pltpu.VMEM((TOK_W,), jnp.float32),
            pltpu.SemaphoreType.DMA,
            pltpu.SemaphoreType.DMA,
        ],
    )
    def dispatch_kernel(x_hbm, p0_hbm, p1_hbm, w0_hbm, w1_hbm,
                        xg_hbm, ws_hbm, xv, i0v, i1v, wv0, wv1, s0, s1):
        wid = lax.axis_index("s") * NC + lax.axis_index("c")
        base = wid * TOK_W
        lx = pltpu.async_copy(x_hbm.at[pl.ds(base, TOK_W)], xv, s0)
        l0 = pltpu.async_copy(p0_hbm.at[pl.ds(base, TOK_W)], i0v, s1)
        l1 = pltpu.async_copy(p1_hbm.at[pl.ds(base, TOK_W)], i1v, s1)
        lw0 = pltpu.async_copy(w0_hbm.at[pl.ds(base, TOK_W)], wv0, s1)
        lw1 = pltpu.async_copy(w1_hbm.at[pl.ds(base, TOK_W)], wv1, s1)
        lx.wait()
        l0.wait()
        l1.wait()
        lw0.wait()
        lw1.wait()
        c0 = pltpu.async_copy(xv, xg_hbm.at[i0v], s0)
        c1 = pltpu.async_copy(xv, xg_hbm.at[i1v], s1)
        cw0 = pltpu.async_copy(wv0, ws_hbm.at[i0v], s0)
        cw1 = pltpu.async_copy(wv1, ws_hbm.at[i1v], s1)
        c0.wait()
        c1.wait()
        cw0.wait()
        cw1.wait()

    @functools.partial(
        pl.kernel,
        out_type=jax.ShapeDtypeStruct((T, D), jnp.float32),
        mesh=mesh,
        scratch_types=[
            [pltpu.VMEM((CT,), jnp.int32) for _ in range(2 * (TOK_W // CT))],
            [pltpu.VMEM((CT, D), jnp.float32) for _ in range(4)],
            [pltpu.SemaphoreType.DMA for _ in range(4)],
        ],
    )
    def combine_kernel(y_hbm, p0_hbm, p1_hbm, out_hbm, idxs, bufs, sems):
        wid = lax.axis_index("s") * NC + lax.axis_index("c")
        nch = TOK_W // CT
        for c in range(nch):
            off = wid * TOK_W + c * CT
            pltpu.sync_copy(p0_hbm.at[pl.ds(off, CT)], idxs[2 * c])
            pltpu.sync_copy(p1_hbm.at[pl.ds(off, CT)], idxs[2 * c + 1])

        def fire(c):
            par = c % 2
            return (
                pltpu.async_copy(y_hbm.at[idxs[2 * c]], bufs[2 * par], sems[2 * par]),
                pltpu.async_copy(y_hbm.at[idxs[2 * c + 1]], bufs[2 * par + 1],
                                 sems[2 * par + 1]),
            )

        pend = fire(0)
        for c in range(nch):
            par = c % 2
            b0 = bufs[2 * par]
            b1 = bufs[2 * par + 1]
            cur = pend
            if c + 1 < nch:
                pend = fire(c + 1)
            cur[0].wait()
            cur[1].wait()

            def row_add(r, carry):
                for j in range(NV):
                    sl = pl.ds(j * 16, 16)
                    b0[r, sl] = b0[r, sl] + b1[r, sl]
                return carry

            lax.fori_loop(0, CT, row_add, 0)
            pltpu.sync_copy(b0, out_hbm.at[pl.ds(wid * TOK_W + c * CT, CT)])

    return dispatch_kernel, combine_kernel


def _mlp_body(g_ref, xg_ref, ws_ref, w1_ref, w3_ref, w2_ref, y_ref):
    xb = xg_ref[...]
    h1 = jnp.dot(xb, w1_ref[0], preferred_element_type=jnp.float32,
                 precision=lax.Precision.DEFAULT)
    h3 = jnp.dot(xb, w3_ref[0], preferred_element_type=jnp.float32,
                 precision=lax.Precision.DEFAULT)
    h = (h1 * jax.nn.sigmoid(h1)) * h3
    y = jnp.dot(h, w2_ref[0], preferred_element_type=jnp.float32,
                precision=lax.Precision.DEFAULT)
    y_ref[...] = y * ws_ref[0]


def _grouped_mlp(g, Xg, ws, W1, W3, W2):
    grid_spec = pltpu.PrefetchScalarGridSpec(
        num_scalar_prefetch=1,
        grid=(NTILES,),
        in_specs=[
            pl.BlockSpec((TM, D), lambda j, gm: (j, 0)),
            pl.BlockSpec((1, TM, 1), lambda j, gm: (j, 0, 0)),
            pl.BlockSpec((1, D, F), lambda j, gm: (gm[j], 0, 0)),
            pl.BlockSpec((1, D, F), lambda j, gm: (gm[j], 0, 0)),
            pl.BlockSpec((1, F, D), lambda j, gm: (gm[j], 0, 0)),
        ],
        out_specs=pl.BlockSpec((TM, D), lambda j, gm: (j, 0)),
    )
    return pl.pallas_call(
        _mlp_body,
        grid_spec=grid_spec,
        out_shape=jax.ShapeDtypeStruct((PP, D), jnp.float32),
    )(g, Xg, ws.reshape(NTILES, TM, 1), W1, W3, W2)


def kernel(x, Wg, bg, W1, W3, W2):
    dispatch_kernel, combine_kernel = _sc_kernels()
    p0, p1, w0, w1, g = _router(x, Wg, bg)
    p0 = p0.reshape(T)
    p1 = p1.reshape(T)
    Xg, ws = dispatch_kernel(x, p0, p1, w0.reshape(T), w1.reshape(T))
    Y = _grouped_mlp(g.reshape(NTILES), Xg, ws, W1, W3, W2)
    return combine_kernel(Y, p0, p1)


# speedup vs baseline: 1.2995x; 1.1568x over previous
import functools

import jax
import jax.numpy as jnp
from jax import lax
from jax.experimental import pallas as pl
from jax.experimental.pallas import tpu as pltpu
from jax.experimental.pallas import tpu_sc as plsc

E = 8
TOPK = 2
D = 768
F = 2048
T = 4096
NPAIR = T * TOPK

TM = 256
NTILES = NPAIR // TM + E
PP = NTILES * TM

BC = 128
NB = T // BC

NC = 2
NS = 16
NW = NC * NS
TOK_W = T // NW
CT = 32
NV = D // 16


def _router_body(x_ref, wg_ref, bg_ref, p0_ref, p1_ref, w0_ref, w1_ref, g_ref,
                 m_scr, cs_scr):
    x = x_ref[...]
    logits = jnp.dot(x, wg_ref[...], preferred_element_type=jnp.float32)
    logits = logits + bg_ref[...]
    mx = jnp.max(logits, axis=1, keepdims=True)
    ex = jnp.exp(logits - mx)
    probs = ex / jnp.sum(ex, axis=1, keepdims=True)

    iota = lax.broadcasted_iota(jnp.int32, (T, E), 1)
    w0 = jnp.max(probs, axis=1, keepdims=True)
    i0 = jnp.min(jnp.where(probs == w0, iota, E), axis=1, keepdims=True)
    probs2 = jnp.where(iota == i0, -jnp.inf, probs)
    w1 = jnp.max(probs2, axis=1, keepdims=True)
    i1 = jnp.min(jnp.where(probs2 == w1, iota, E), axis=1, keepdims=True)
    s = w0 + w1
    w0 = w0 / s
    w1 = w1 / s

    m_scr[...] = ((iota == i0) | (iota == i1)).astype(jnp.float32)

    r = lax.broadcasted_iota(jnp.int32, (BC, BC), 0)
    c = lax.broadcasted_iota(jnp.int32, (BC, BC), 1)
    ltri = (c < r).astype(jnp.float32)

    def blk(b, off):
        rows = m_scr[pl.ds(b * BC, BC), :]
        cs_scr[pl.ds(b * BC, BC), :] = (
            jnp.dot(ltri, rows, preferred_element_type=jnp.float32) + off
        )
        return off + jnp.sum(rows, axis=0, keepdims=True)

    cnt = lax.fori_loop(0, NB, blk, jnp.zeros((1, E), jnp.float32))

    pc = jnp.ceil(cnt / TM) * TM
    re8 = lax.broadcasted_iota(jnp.int32, (E, E), 0)
    ce8 = lax.broadcasted_iota(jnp.int32, (E, E), 1)
    ut8 = (re8 < ce8).astype(jnp.float32)
    po = jnp.dot(pc, ut8, preferred_element_type=jnp.float32)

    base = po + cs_scr[...]
    pos0 = jnp.sum(jnp.where(iota == i0, base, 0.0), axis=1, keepdims=True)
    pos1 = jnp.sum(jnp.where(iota == i1, base, 0.0), axis=1, keepdims=True)

    p0_ref[...] = pos0.astype(jnp.int32).reshape(T // 128, 128)
    p1_ref[...] = pos1.astype(jnp.int32).reshape(T // 128, 128)
    w0_ref[...] = w0.reshape(T // 128, 128)
    w1_ref[...] = w1.reshape(T // 128, 128)

    jt = lax.broadcasted_iota(jnp.int32, (NTILES, E), 0).astype(jnp.float32) * TM
    ends = po + pc
    gmap = jnp.sum((jt >= ends).astype(jnp.int32), axis=1, keepdims=True)
    g_ref[...] = jnp.minimum(gmap, E - 1)


def _router(x, Wg, bg):
    return pl.pallas_call(
        _router_body,
        out_shape=(
            jax.ShapeDtypeStruct((T // 128, 128), jnp.int32),
            jax.ShapeDtypeStruct((T // 128, 128), jnp.int32),
            jax.ShapeDtypeStruct((T // 128, 128), jnp.float32),
            jax.ShapeDtypeStruct((T // 128, 128), jnp.float32),
            jax.ShapeDtypeStruct((NTILES, 1), jnp.int32),
        ),
        scratch_shapes=[
            pltpu.VMEM((T, E), jnp.float32),
            pltpu.VMEM((T, E), jnp.float32),
        ],
    )(x, Wg, bg.reshape(1, E))


@functools.cache
def _sc_kernels():
    mesh = plsc.VectorSubcoreMesh(
        core_axis_name="c", subcore_axis_name="s", num_cores=NC, num_subcores=NS
    )

    @functools.partial(
        pl.kernel,
        out_type=jax.ShapeDtypeStruct((PP, D), jnp.float32),
        mesh=mesh,
        scratch_types=[
            pltpu.VMEM((TOK_W, D), jnp.float32),
            pltpu.VMEM((TOK_W,), jnp.int32),
            pltpu.VMEM((TOK_W,), jnp.int32),
            pltpu.SemaphoreType.DMA,
            pltpu.SemaphoreType.DMA,
        ],
    )
    def dispatch_kernel(x_hbm, p0_hbm, p1_hbm, xg_hbm, xv, i0v, i1v, s0, s1):
        wid = lax.axis_index("s") * NC + lax.axis_index("c")
        base = wid * TOK_W
        lx = pltpu.async_copy(x_hbm.at[pl.ds(base, TOK_W)], xv, s0)
        l0 = pltpu.async_copy(p0_hbm.at[pl.ds(base, TOK_W)], i0v, s1)
        l1 = pltpu.async_copy(p1_hbm.at[pl.ds(base, TOK_W)], i1v, s1)
        lx.wait()
        l0.wait()
        l1.wait()
        c0 = pltpu.async_copy(xv, xg_hbm.at[i0v], s0)
        c1 = pltpu.async_copy(xv, xg_hbm.at[i1v], s1)
        c0.wait()
        c1.wait()

    @functools.partial(
        pl.kernel,
        out_type=jax.ShapeDtypeStruct((T, D), jnp.float32),
        mesh=mesh,
        scratch_types=[
            [pltpu.VMEM((CT,), jnp.int32) for _ in range(2 * (TOK_W // CT))],
            [pltpu.VMEM((CT + 16,), jnp.float32) for _ in range(2 * (TOK_W // CT))],
            [pltpu.VMEM((CT, D), jnp.float32) for _ in range(4)],
            [pltpu.SemaphoreType.DMA for _ in range(4)],
        ],
    )
    def combine_kernel(y_hbm, p0_hbm, p1_hbm, w0_hbm, w1_hbm, out_hbm,
                       idxs, wchunks, bufs, sems):
        wid = lax.axis_index("s") * NC + lax.axis_index("c")
        nch = TOK_W // CT
        for c in range(nch):
            off = wid * TOK_W + c * CT
            pltpu.sync_copy(p0_hbm.at[pl.ds(off, CT)], idxs[2 * c])
            pltpu.sync_copy(p1_hbm.at[pl.ds(off, CT)], idxs[2 * c + 1])
            pltpu.sync_copy(w0_hbm.at[pl.ds(off, CT)], wchunks[2 * c].at[pl.ds(0, CT)])
            pltpu.sync_copy(w1_hbm.at[pl.ds(off, CT)], wchunks[2 * c + 1].at[pl.ds(0, CT)])

        def fire(c):
            par = c % 2
            return (
                pltpu.async_copy(y_hbm.at[idxs[2 * c]], bufs[2 * par], sems[2 * par]),
                pltpu.async_copy(y_hbm.at[idxs[2 * c + 1]], bufs[2 * par + 1],
                                 sems[2 * par + 1]),
            )

        pend = fire(0)
        for c in range(nch):
            par = c % 2
            b0 = bufs[2 * par]
            b1 = bufs[2 * par + 1]
            wc0 = wchunks[2 * c]
            wc1 = wchunks[2 * c + 1]
            cur = pend
            if c + 1 < nch:
                pend = fire(c + 1)
            cur[0].wait()
            cur[1].wait()

            def row_add(r, carry):
                w0s = wc0[pl.ds(r, 16)][0]
                w1s = wc1[pl.ds(r, 16)][0]
                for j in range(NV):
                    sl = pl.ds(j * 16, 16)
                    b0[r, sl] = b0[r, sl] * w0s + b1[r, sl] * w1s
                return carry

            lax.fori_loop(0, CT, row_add, 0)
            pltpu.sync_copy(b0, out_hbm.at[pl.ds(wid * TOK_W + c * CT, CT)])

    return dispatch_kernel, combine_kernel


def _mlp_body(g_ref, xg_ref, w1_ref, w3_ref, w2_ref, y_ref):
    xb = xg_ref[...]
    h1 = jnp.dot(xb, w1_ref[0], preferred_element_type=jnp.float32,
                 precision=lax.Precision.DEFAULT)
    h3 = jnp.dot(xb, w3_ref[0], preferred_element_type=jnp.float32,
                 precision=lax.Precision.DEFAULT)
    h = (h1 * jax.nn.sigmoid(h1)) * h3
    y_ref[...] = jnp.dot(h, w2_ref[0], preferred_element_type=jnp.float32,
                         precision=lax.Precision.DEFAULT)


def _grouped_mlp(g, Xg, W1, W3, W2):
    grid_spec = pltpu.PrefetchScalarGridSpec(
        num_scalar_prefetch=1,
        grid=(NTILES,),
        in_specs=[
            pl.BlockSpec((TM, D), lambda j, gm: (j, 0)),
            pl.BlockSpec((1, D, F), lambda j, gm: (gm[j], 0, 0)),
            pl.BlockSpec((1, D, F), lambda j, gm: (gm[j], 0, 0)),
            pl.BlockSpec((1, F, D), lambda j, gm: (gm[j], 0, 0)),
        ],
        out_specs=pl.BlockSpec((TM, D), lambda j, gm: (j, 0)),
    )
    return pl.pallas_call(
        _mlp_body,
        grid_spec=grid_spec,
        out_shape=jax.ShapeDtypeStruct((PP, D), jnp.float32),
    )(g, Xg, W1, W3, W2)


def kernel(x, Wg, bg, W1, W3, W2):
    dispatch_kernel, combine_kernel = _sc_kernels()
    p0, p1, w0, w1, g = _router(x, Wg, bg)
    p0 = p0.reshape(T)
    p1 = p1.reshape(T)
    Xg = dispatch_kernel(x, p0, p1)
    Y = _grouped_mlp(g.reshape(NTILES), Xg, W1, W3, W2)
    return combine_kernel(Y, p0, p1, w0.reshape(T), w1.reshape(T))


# speedup vs baseline: 1.3467x; 1.0363x over previous
import functools

import jax
import jax.numpy as jnp
from jax import lax
from jax.experimental import pallas as pl
from jax.experimental.pallas import tpu as pltpu
from jax.experimental.pallas import tpu_sc as plsc

E = 8
TOPK = 2
D = 768
F = 2048
T = 4096
NPAIR = T * TOPK

TM = 256
NTILES = NPAIR // TM + E
PP = NTILES * TM

BC = 128
NB = T // BC

NC = 2
NS = 16
NW = NC * NS
TOK_W = T // NW
CT = 32
NV = D // 16


def _router_body(x_ref, wg_ref, bg_ref, p0_ref, p1_ref, w0_ref, w1_ref, g_ref,
                 m_scr, cs_scr):
    x = x_ref[...]
    logits = jnp.dot(x, wg_ref[...], preferred_element_type=jnp.float32)
    logits = logits + bg_ref[...]
    mx = jnp.max(logits, axis=1, keepdims=True)
    ex = jnp.exp(logits - mx)
    probs = ex / jnp.sum(ex, axis=1, keepdims=True)

    iota = lax.broadcasted_iota(jnp.int32, (T, E), 1)
    w0 = jnp.max(probs, axis=1, keepdims=True)
    i0 = jnp.min(jnp.where(probs == w0, iota, E), axis=1, keepdims=True)
    probs2 = jnp.where(iota == i0, -jnp.inf, probs)
    w1 = jnp.max(probs2, axis=1, keepdims=True)
    i1 = jnp.min(jnp.where(probs2 == w1, iota, E), axis=1, keepdims=True)
    s = w0 + w1
    w0 = w0 / s
    w1 = w1 / s

    m_scr[...] = ((iota == i0) | (iota == i1)).astype(jnp.float32)

    r = lax.broadcasted_iota(jnp.int32, (BC, BC), 0)
    c = lax.broadcasted_iota(jnp.int32, (BC, BC), 1)
    ltri = (c < r).astype(jnp.float32)

    def blk(b, off):
        rows = m_scr[pl.ds(b * BC, BC), :]
        cs_scr[pl.ds(b * BC, BC), :] = (
            jnp.dot(ltri, rows, preferred_element_type=jnp.float32) + off
        )
        return off + jnp.sum(rows, axis=0, keepdims=True)

    cnt = lax.fori_loop(0, NB, blk, jnp.zeros((1, E), jnp.float32))

    pc = jnp.ceil(cnt / TM) * TM
    re8 = lax.broadcasted_iota(jnp.int32, (E, E), 0)
    ce8 = lax.broadcasted_iota(jnp.int32, (E, E), 1)
    ut8 = (re8 < ce8).astype(jnp.float32)
    po = jnp.dot(pc, ut8, preferred_element_type=jnp.float32)

    base = po + cs_scr[...]
    pos0 = jnp.sum(jnp.where(iota == i0, base, 0.0), axis=1, keepdims=True)
    pos1 = jnp.sum(jnp.where(iota == i1, base, 0.0), axis=1, keepdims=True)

    p0_ref[...] = pos0.astype(jnp.int32).reshape(T // 128, 128)
    p1_ref[...] = pos1.astype(jnp.int32).reshape(T // 128, 128)
    w0_ref[...] = w0.reshape(T // 128, 128)
    w1_ref[...] = w1.reshape(T // 128, 128)

    jt = lax.broadcasted_iota(jnp.int32, (NTILES, E), 0).astype(jnp.float32) * TM
    ends = po + pc
    g_ref[...] = jnp.sum((jt >= ends).astype(jnp.int32), axis=1, keepdims=True)


def _router(x, Wg, bg):
    return pl.pallas_call(
        _router_body,
        out_shape=(
            jax.ShapeDtypeStruct((T // 128, 128), jnp.int32),
            jax.ShapeDtypeStruct((T // 128, 128), jnp.int32),
            jax.ShapeDtypeStruct((T // 128, 128), jnp.float32),
            jax.ShapeDtypeStruct((T // 128, 128), jnp.float32),
            jax.ShapeDtypeStruct((NTILES, 1), jnp.int32),
        ),
        scratch_shapes=[
            pltpu.VMEM((T, E), jnp.float32),
            pltpu.VMEM((T, E), jnp.float32),
        ],
    )(x, Wg, bg.reshape(1, E))


@functools.cache
def _sc_kernels():
    mesh = plsc.VectorSubcoreMesh(
        core_axis_name="c", subcore_axis_name="s", num_cores=NC, num_subcores=NS
    )

    @functools.partial(
        pl.kernel,
        out_type=jax.ShapeDtypeStruct((PP, D), jnp.float32),
        mesh=mesh,
        scratch_types=[
            pltpu.VMEM((TOK_W, D), jnp.float32),
            pltpu.VMEM((TOK_W,), jnp.int32),
            pltpu.VMEM((TOK_W,), jnp.int32),
            pltpu.SemaphoreType.DMA,
            pltpu.SemaphoreType.DMA,
        ],
    )
    def dispatch_kernel(x_hbm, p0_hbm, p1_hbm, xg_hbm, xv, i0v, i1v, s0, s1):
        wid = lax.axis_index("s") * NC + lax.axis_index("c")
        base = wid * TOK_W
        lx = pltpu.async_copy(x_hbm.at[pl.ds(base, TOK_W)], xv, s0)
        l0 = pltpu.async_copy(p0_hbm.at[pl.ds(base, TOK_W)], i0v, s1)
        l1 = pltpu.async_copy(p1_hbm.at[pl.ds(base, TOK_W)], i1v, s1)
        lx.wait()
        l0.wait()
        l1.wait()
        c0 = pltpu.async_copy(xv, xg_hbm.at[i0v], s0)
        c1 = pltpu.async_copy(xv, xg_hbm.at[i1v], s1)
        c0.wait()
        c1.wait()

    @functools.partial(
        pl.kernel,
        out_type=jax.ShapeDtypeStruct((T, D), jnp.float32),
        mesh=mesh,
        scratch_types=[
            [pltpu.VMEM((CT,), jnp.int32) for _ in range(2 * (TOK_W // CT))],
            [pltpu.VMEM((CT + 16,), jnp.float32) for _ in range(2 * (TOK_W // CT))],
            [pltpu.VMEM((CT, D), jnp.float32) for _ in range(4)],
            [pltpu.SemaphoreType.DMA for _ in range(4)],
        ],
    )
    def combine_kernel(y_hbm, p0_hbm, p1_hbm, w0_hbm, w1_hbm, out_hbm,
                       idxs, wchunks, bufs, sems):
        wid = lax.axis_index("s") * NC + lax.axis_index("c")
        nch = TOK_W // CT
        for c in range(nch):
            off = wid * TOK_W + c * CT
            pltpu.sync_copy(p0_hbm.at[pl.ds(off, CT)], idxs[2 * c])
            pltpu.sync_copy(p1_hbm.at[pl.ds(off, CT)], idxs[2 * c + 1])
            pltpu.sync_copy(w0_hbm.at[pl.ds(off, CT)], wchunks[2 * c].at[pl.ds(0, CT)])
            pltpu.sync_copy(w1_hbm.at[pl.ds(off, CT)], wchunks[2 * c + 1].at[pl.ds(0, CT)])

        def fire(c):
            par = c % 2
            return (
                pltpu.async_copy(y_hbm.at[idxs[2 * c]], bufs[2 * par], sems[2 * par]),
                pltpu.async_copy(y_hbm.at[idxs[2 * c + 1]], bufs[2 * par + 1],
                                 sems[2 * par + 1]),
            )

        pend = fire(0)
        for c in range(nch):
            par = c % 2
            b0 = bufs[2 * par]
            b1 = bufs[2 * par + 1]
            wc0 = wchunks[2 * c]
            wc1 = wchunks[2 * c + 1]
            cur = pend
            if c + 1 < nch:
                pend = fire(c + 1)
            cur[0].wait()
            cur[1].wait()

            def row_add(r, carry):
                w0s = wc0[pl.ds(r, 16)][0]
                w1s = wc1[pl.ds(r, 16)][0]
                for j in range(NV):
                    sl = pl.ds(j * 16, 16)
                    b0[r, sl] = b0[r, sl] * w0s + b1[r, sl] * w1s
                return carry

            lax.fori_loop(0, CT, row_add, 0)
            pltpu.sync_copy(b0, out_hbm.at[pl.ds(wid * TOK_W + c * CT, CT)])

    return dispatch_kernel, combine_kernel


def _mlp_body(g_ref, xg_ref, w1_ref, w3_ref, w2_ref, y_ref):
    @pl.when(g_ref[pl.program_id(0)] < E)
    def _():
        xb = xg_ref[...]
        h1 = jnp.dot(xb, w1_ref[0], preferred_element_type=jnp.float32,
                     precision=lax.Precision.DEFAULT)
        h3 = jnp.dot(xb, w3_ref[0], preferred_element_type=jnp.float32,
                     precision=lax.Precision.DEFAULT)
        h = (h1 * jax.nn.sigmoid(h1)) * h3
        y_ref[...] = jnp.dot(h, w2_ref[0], preferred_element_type=jnp.float32,
                             precision=lax.Precision.DEFAULT)


def _grouped_mlp(g, Xg, W1, W3, W2):
    grid_spec = pltpu.PrefetchScalarGridSpec(
        num_scalar_prefetch=1,
        grid=(NTILES,),
        in_specs=[
            pl.BlockSpec((TM, D), lambda j, gm: (j, 0)),
            pl.BlockSpec((1, D, F), lambda j, gm: (jnp.minimum(gm[j], E - 1), 0, 0)),
            pl.BlockSpec((1, D, F), lambda j, gm: (jnp.minimum(gm[j], E - 1), 0, 0)),
            pl.BlockSpec((1, F, D), lambda j, gm: (jnp.minimum(gm[j], E - 1), 0, 0)),
        ],
        out_specs=pl.BlockSpec((TM, D), lambda j, gm: (j, 0)),
    )
    return pl.pallas_call(
        _mlp_body,
        grid_spec=grid_spec,
        out_shape=jax.ShapeDtypeStruct((PP, D), jnp.float32),
    )(g, Xg, W1, W3, W2)


def kernel(x, Wg, bg, W1, W3, W2):
    dispatch_kernel, combine_kernel = _sc_kernels()
    p0, p1, w0, w1, g = _router(x, Wg, bg)
    p0 = p0.reshape(T)
    p1 = p1.reshape(T)
    Xg = dispatch_kernel(x, p0, p1)
    Y = _grouped_mlp(g.reshape(NTILES), Xg, W1, W3, W2)
    return combine_kernel(Y, p0, p1, w0.reshape(T), w1.reshape(T))
